# baseline (speedup 1.0000x reference)
import functools

import jax
import jax.numpy as jnp
from jax import lax
from jax.experimental import pallas as pl
from jax.experimental.pallas import tpu as pltpu
from jax.experimental.pallas import tpu_sc as plsc

BATCH = 16384
EMBED_DIM = 64
PAIR = 2 * EMBED_DIM
NUM_WORKERS = 32
BPW = BATCH // NUM_WORKERS
GW = 128
MLP_BLOCK = 2048


def _sc_gather4(packed, pidxs):
    mesh = plsc.VectorSubcoreMesh(core_axis_name="core", subcore_axis_name="subcore")
    out_type = [jax.ShapeDtypeStruct((BATCH, PAIR), jnp.float32)] * 4

    @functools.partial(
        pl.kernel, out_type=out_type, mesh=mesh,
        scratch_types=[pltpu.VMEM((BPW,), jnp.int32),
                       pltpu.VMEM((BPW, PAIR), jnp.float32),
                       pltpu.SemaphoreType.DMA])
    def gather_kernel(t0, t1, t2, t3, i0, i1, i2, i3,
                      o0, o1, o2, o3, idx_v, rows, sem):
        cid = lax.axis_index("core")
        sid = lax.axis_index("subcore")
        base = (sid * 2 + cid) * BPW
        for tbl, idx, out in ((t0, i0, o0), (t1, i1, o1),
                              (t2, i2, o2), (t3, i3, o3)):
            pltpu.sync_copy(idx.at[pl.ds(base, BPW)], idx_v)
            copies = []
            for j in range(BPW // GW):
                copies.append(pltpu.async_copy(
                    tbl.at[idx_v.at[pl.ds(j * GW, GW)]],
                    rows.at[pl.ds(j * GW, GW), :], sem))
            for c in copies:
                c.wait()
            pltpu.sync_copy(rows, out.at[pl.ds(base, BPW), :])

    return gather_kernel(*packed, *pidxs)


def _mlp_body(u_ref, r_ref, i_ref, n_ref, pu_ref, pr_ref, pi_ref, pn_ref,
              w1_ref, b1_ref, w2_ref, b2_ref, o_ref):
    halves = []
    for x_ref, p_ref in ((u_ref, pu_ref), (r_ref, pr_ref),
                         (i_ref, pi_ref), (n_ref, pn_ref)):
        x = x_ref[...]
        odd = p_ref[...] == 1
        halves.append(jnp.where(odd, x[:, EMBED_DIM:], x[:, :EMBED_DIM]))
    x = jnp.concatenate(halves, axis=1)
    h = jnp.dot(x, w1_ref[...], preferred_element_type=jnp.float32) + b1_ref[...]
    h = jnp.maximum(h, 0.0)
    o_ref[...] = jnp.dot(h, w2_ref[...], preferred_element_type=jnp.float32) + b2_ref[...]


def _tc_mlp(pairs, parities, W1, b1, W2, b2):
    d4, d2, d1 = 4 * EMBED_DIM, 2 * EMBED_DIM, EMBED_DIM
    pair_spec = pl.BlockSpec((MLP_BLOCK, PAIR), lambda g: (g, 0))
    par_spec = pl.BlockSpec((MLP_BLOCK, 1), lambda g: (g, 0))
    return pl.pallas_call(
        _mlp_body,
        grid=(BATCH // MLP_BLOCK,),
        in_specs=[
            pair_spec, pair_spec, pair_spec, pair_spec,
            par_spec, par_spec, par_spec, par_spec,
            pl.BlockSpec((d4, d2), lambda g: (0, 0)),
            pl.BlockSpec((1, d2), lambda g: (0, 0)),
            pl.BlockSpec((d2, d1), lambda g: (0, 0)),
            pl.BlockSpec((1, d1), lambda g: (0, 0)),
        ],
        out_specs=pl.BlockSpec((MLP_BLOCK, d1), lambda g: (g, 0)),
        out_shape=jax.ShapeDtypeStruct((BATCH, d1), jnp.float32),
    )(*pairs, *parities, W1, b1.reshape(1, d2), W2, b2.reshape(1, d1))


@jax.jit
def kernel(user_idx, recipe_idx, ingredient_idx, nutrition_idx,
           user_table, recipe_table, ingredient_table, nutrition_table,
           W1, b1, W2, b2):
    idxs = [x.astype(jnp.int32)
            for x in (user_idx, recipe_idx, ingredient_idx, nutrition_idx)]
    pidxs = [x >> 1 for x in idxs]
    parities = [(x & 1).reshape(BATCH, 1) for x in idxs]
    packed = [t.reshape(-1, PAIR)
              for t in (user_table, recipe_table, ingredient_table,
                        nutrition_table)]
    pairs = _sc_gather4(packed, pidxs)
    return _tc_mlp(pairs, parities, W1, b1, W2, b2)


# speedup vs baseline: 1.4378x; 1.4378x over previous
import functools

import jax
import jax.numpy as jnp
from jax import lax
from jax.experimental import pallas as pl
from jax.experimental.pallas import tpu as pltpu
from jax.experimental.pallas import tpu_sc as plsc

BATCH = 16384
EMBED_DIM = 64
HALF = EMBED_DIM // 2
PACK_LANES = 4096
PACK_ROWS = PACK_LANES // 4
NUM_WORKERS = 32
BPW = BATCH // NUM_WORKERS
GW = 128
MLP_BLOCK = 2048
TOPMASK = -65536


def _rne_bf16_bits(x):
    b = lax.bitcast_convert_type(x, jnp.int32)
    r = b + 0x7FFF + (lax.shift_right_logical(b, 16) & 1)
    return r & TOPMASK


def _pack_body(x_ref, eye_ref, o_ref):
    x = x_ref[...].astype(jnp.bfloat16)
    eye = eye_ref[...]
    dn = (((0,), (0,)), ((), ()))
    quarters = []
    for q in range(4):
        xt = lax.dot_general(x[:, q * PACK_ROWS:(q + 1) * PACK_ROWS], eye, dn,
                             preferred_element_type=jnp.float32)
        lo = lax.shift_right_logical(_rne_bf16_bits(xt[:, :HALF]), 16)
        hi = _rne_bf16_bits(xt[:, HALF:])
        quarters.append(lo | hi)
    packed = jnp.concatenate(quarters, axis=1)
    o_ref[...] = lax.bitcast_convert_type(packed, jnp.float32)


def _tc_pack(tT, eye):
    n = tT.shape[1]
    grid = pl.cdiv(n, PACK_LANES)
    return pl.pallas_call(
        _pack_body,
        grid=(grid,),
        in_specs=[pl.BlockSpec((EMBED_DIM, PACK_LANES), lambda g: (0, g)),
                  pl.BlockSpec((EMBED_DIM, EMBED_DIM), lambda g: (0, 0))],
        out_specs=pl.BlockSpec((PACK_ROWS, 128), lambda g: (g, 0)),
        out_shape=jax.ShapeDtypeStruct((grid * PACK_ROWS, 128), jnp.float32),
    )(tT, eye)


def _sc_gather4(packed, pidxs):
    mesh = plsc.VectorSubcoreMesh(core_axis_name="core", subcore_axis_name="subcore")
    out_type = [jax.ShapeDtypeStruct((BATCH, 128), jnp.float32)] * 4

    @functools.partial(
        pl.kernel, out_type=out_type, mesh=mesh,
        scratch_types=[pltpu.VMEM((BPW,), jnp.int32),
                       pltpu.VMEM((BPW, 128), jnp.float32),
                       pltpu.SemaphoreType.DMA])
    def gather_kernel(t0, t1, t2, t3, i0, i1, i2, i3,
                      o0, o1, o2, o3, idx_v, rows, sem):
        cid = lax.axis_index("core")
        sid = lax.axis_index("subcore")
        base = (sid * 2 + cid) * BPW
        for tbl, idx, out in ((t0, i0, o0), (t1, i1, o1),
                              (t2, i2, o2), (t3, i3, o3)):
            pltpu.sync_copy(idx.at[pl.ds(base, BPW)], idx_v)
            copies = []
            for j in range(BPW // GW):
                copies.append(pltpu.async_copy(
                    tbl.at[idx_v.at[pl.ds(j * GW, GW)]],
                    rows.at[pl.ds(j * GW, GW), :], sem))
            for c in copies:
                c.wait()
            pltpu.sync_copy(rows, out.at[pl.ds(base, BPW), :])

    return gather_kernel(*packed, *pidxs)


def _mlp_body(u_ref, r_ref, i_ref, n_ref, qu_ref, qr_ref, qi_ref, qn_ref,
              w1_ref, b1_ref, w2_ref, b2_ref, o_ref):
    embs = []
    for x_ref, q_ref in ((u_ref, qu_ref), (r_ref, qr_ref),
                         (i_ref, qi_ref), (n_ref, qn_ref)):
        v = lax.bitcast_convert_type(x_ref[...], jnp.int32)
        q = q_ref[...]
        sel = v[:, :HALF]
        for k in (1, 2, 3):
            sel = jnp.where(q == k, v[:, k * HALF:(k + 1) * HALF], sel)
        lo = lax.bitcast_convert_type(lax.shift_left(sel, 16), jnp.float32)
        hi = lax.bitcast_convert_type(sel & TOPMASK, jnp.float32)
        embs.append(lo)
        embs.append(hi)
    x = jnp.concatenate(embs, axis=1).astype(jnp.bfloat16)
    w1 = w1_ref[...].astype(jnp.bfloat16)
    h = jnp.dot(x, w1, preferred_element_type=jnp.float32) + b1_ref[...]
    h = jnp.maximum(h, 0.0).astype(jnp.bfloat16)
    w2 = w2_ref[...].astype(jnp.bfloat16)
    o_ref[...] = jnp.dot(h, w2, preferred_element_type=jnp.float32) + b2_ref[...]


def _tc_mlp(pairs, quarters, W1, b1, W2, b2):
    d4, d2, d1 = 4 * EMBED_DIM, 2 * EMBED_DIM, EMBED_DIM
    row_spec = pl.BlockSpec((MLP_BLOCK, 128), lambda g: (g, 0))
    q_spec = pl.BlockSpec((MLP_BLOCK, 1), lambda g: (g, 0))
    return pl.pallas_call(
        _mlp_body,
        grid=(BATCH // MLP_BLOCK,),
        in_specs=[
            row_spec, row_spec, row_spec, row_spec,
            q_spec, q_spec, q_spec, q_spec,
            pl.BlockSpec((d4, d2), lambda g: (0, 0)),
            pl.BlockSpec((1, d2), lambda g: (0, 0)),
            pl.BlockSpec((d2, d1), lambda g: (0, 0)),
            pl.BlockSpec((1, d1), lambda g: (0, 0)),
        ],
        out_specs=pl.BlockSpec((MLP_BLOCK, d1), lambda g: (g, 0)),
        out_shape=jax.ShapeDtypeStruct((BATCH, d1), jnp.float32),
    )(*pairs, *quarters, W1, b1.reshape(1, d2), W2, b2.reshape(1, d1))


@jax.jit
def kernel(user_idx, recipe_idx, ingredient_idx, nutrition_idx,
           user_table, recipe_table, ingredient_table, nutrition_table,
           W1, b1, W2, b2):
    eye = jnp.eye(EMBED_DIM, dtype=jnp.bfloat16)
    pidxs, quarters, packed = [], [], []
    for idx, tbl in ((user_idx, user_table), (recipe_idx, recipe_table),
                     (ingredient_idx, ingredient_table),
                     (nutrition_idx, nutrition_table)):
        r = idx.astype(jnp.int32)
        blk = r // PACK_LANES
        off = r % PACK_LANES
        pidxs.append(blk * PACK_ROWS + off % PACK_ROWS)
        quarters.append((off // PACK_ROWS).reshape(BATCH, 1))
        packed.append(_tc_pack(tbl.T, eye))
    pairs = _sc_gather4(packed, pidxs)
    return _tc_mlp(pairs, quarters, W1, b1, W2, b2)


# speedup vs baseline: 2.1129x; 1.4695x over previous
import functools

import jax
import jax.numpy as jnp
from jax import lax
from jax.experimental import pallas as pl
from jax.experimental.pallas import tpu as pltpu
from jax.experimental.pallas import tpu_sc as plsc

BATCH = 16384
EMBED_DIM = 64
HALF = EMBED_DIM // 2
PACK_LANES = 8192
PACK_ROWS = PACK_LANES // 2
NUM_WORKERS = 32
BPW = BATCH // NUM_WORKERS
GW = 128
MLP_BLOCK = 2048
def _pack_body(x_ref, eye_ref, o_ref):
    x = x_ref[...].astype(jnp.bfloat16)
    eye = eye_ref[...]
    dn = (((0,), (0,)), ((), ()))
    xt = lax.dot_general(x, eye, dn, preferred_element_type=jnp.float32)
    o_ref[...] = jnp.concatenate([xt[:PACK_ROWS], xt[PACK_ROWS:]], axis=1)


def _tc_pack(tT, eye):
    n = tT.shape[1]
    grid = pl.cdiv(n, PACK_LANES)
    return pl.pallas_call(
        _pack_body,
        grid=(grid,),
        in_specs=[pl.BlockSpec((EMBED_DIM, PACK_LANES), lambda g: (0, g)),
                  pl.BlockSpec((EMBED_DIM, EMBED_DIM), lambda g: (0, 0))],
        out_specs=pl.BlockSpec((PACK_ROWS, 128), lambda g: (g, 0)),
        out_shape=jax.ShapeDtypeStruct((grid * PACK_ROWS, 128), jnp.float32),
    )(tT, eye)


def _sc_gather4(packed, pidxs):
    mesh = plsc.VectorSubcoreMesh(core_axis_name="core", subcore_axis_name="subcore")
    out_type = [jax.ShapeDtypeStruct((BATCH, 128), jnp.float32)] * 4

    @functools.partial(
        pl.kernel, out_type=out_type, mesh=mesh,
        scratch_types=[pltpu.VMEM((BPW,), jnp.int32),
                       pltpu.VMEM((BPW, 128), jnp.float32),
                       pltpu.SemaphoreType.DMA])
    def gather_kernel(t0, t1, t2, t3, i0, i1, i2, i3,
                      o0, o1, o2, o3, idx_v, rows, sem):
        cid = lax.axis_index("core")
        sid = lax.axis_index("subcore")
        base = (sid * 2 + cid) * BPW
        for tbl, idx, out in ((t0, i0, o0), (t1, i1, o1),
                              (t2, i2, o2), (t3, i3, o3)):
            pltpu.sync_copy(idx.at[pl.ds(base, BPW)], idx_v)
            copies = []
            for j in range(BPW // GW):
                copies.append(pltpu.async_copy(
                    tbl.at[idx_v.at[pl.ds(j * GW, GW)]],
                    rows.at[pl.ds(j * GW, GW), :], sem))
            for c in copies:
                c.wait()
            pltpu.sync_copy(rows, out.at[pl.ds(base, BPW), :])

    return gather_kernel(*packed, *pidxs)


def _mlp_body(u_ref, r_ref, i_ref, n_ref, qu_ref, qr_ref, qi_ref, qn_ref,
              w1_ref, b1_ref, w2_ref, b2_ref, o_ref):
    embs = []
    for x_ref, q_ref in ((u_ref, qu_ref), (r_ref, qr_ref),
                         (i_ref, qi_ref), (n_ref, qn_ref)):
        v = x_ref[...]
        odd = q_ref[...] == 1
        embs.append(jnp.where(odd, v[:, EMBED_DIM:], v[:, :EMBED_DIM]))
    x = jnp.concatenate(embs, axis=1).astype(jnp.bfloat16)
    w1 = w1_ref[...].astype(jnp.bfloat16)
    h = jnp.dot(x, w1, preferred_element_type=jnp.float32) + b1_ref[...]
    h = jnp.maximum(h, 0.0).astype(jnp.bfloat16)
    w2 = w2_ref[...].astype(jnp.bfloat16)
    o_ref[...] = jnp.dot(h, w2, preferred_element_type=jnp.float32) + b2_ref[...]


def _tc_mlp(pairs, quarters, W1, b1, W2, b2):
    d4, d2, d1 = 4 * EMBED_DIM, 2 * EMBED_DIM, EMBED_DIM
    row_spec = pl.BlockSpec((MLP_BLOCK, 128), lambda g: (g, 0))
    q_spec = pl.BlockSpec((MLP_BLOCK, 1), lambda g: (g, 0))
    return pl.pallas_call(
        _mlp_body,
        grid=(BATCH // MLP_BLOCK,),
        in_specs=[
            row_spec, row_spec, row_spec, row_spec,
            q_spec, q_spec, q_spec, q_spec,
            pl.BlockSpec((d4, d2), lambda g: (0, 0)),
            pl.BlockSpec((1, d2), lambda g: (0, 0)),
            pl.BlockSpec((d2, d1), lambda g: (0, 0)),
            pl.BlockSpec((1, d1), lambda g: (0, 0)),
        ],
        out_specs=pl.BlockSpec((MLP_BLOCK, d1), lambda g: (g, 0)),
        out_shape=jax.ShapeDtypeStruct((BATCH, d1), jnp.float32),
    )(*pairs, *quarters, W1, b1.reshape(1, d2), W2, b2.reshape(1, d1))


@jax.jit
def kernel(user_idx, recipe_idx, ingredient_idx, nutrition_idx,
           user_table, recipe_table, ingredient_table, nutrition_table,
           W1, b1, W2, b2):
    eye = jnp.eye(EMBED_DIM, dtype=jnp.bfloat16)
    pidxs, quarters, packed = [], [], []
    for idx, tbl in ((user_idx, user_table), (recipe_idx, recipe_table),
                     (ingredient_idx, ingredient_table),
                     (nutrition_idx, nutrition_table)):
        r = idx.astype(jnp.int32)
        blk = r // PACK_LANES
        off = r % PACK_LANES
        pidxs.append(blk * PACK_ROWS + off % PACK_ROWS)
        quarters.append((off // PACK_ROWS).reshape(BATCH, 1))
        packed.append(_tc_pack(tbl.T, eye))
    pairs = _sc_gather4(packed, pidxs)
    return _tc_mlp(pairs, quarters, W1, b1, W2, b2)


# speedup vs baseline: 2.4272x; 1.1488x over previous
import functools

import jax
import jax.numpy as jnp
from jax import lax
from jax.experimental import pallas as pl
from jax.experimental.pallas import tpu as pltpu
from jax.experimental.pallas import tpu_sc as plsc

BATCH = 16384
EMBED_DIM = 64
HALF = EMBED_DIM // 2
PACK_LANES = 16384
PACK_ROWS = PACK_LANES // 2
NUM_WORKERS = 32
BPW = BATCH // NUM_WORKERS
GW = 128
MLP_BLOCK = 2048
def _pack_body(x_ref, eye_ref, o_ref):
    x = x_ref[...].astype(jnp.bfloat16)
    eye = eye_ref[...]
    dn = (((0,), (0,)), ((), ()))
    xt = lax.dot_general(x, eye, dn, preferred_element_type=jnp.float32)
    o_ref[...] = jnp.concatenate([xt[:PACK_ROWS], xt[PACK_ROWS:]], axis=1)


def _tc_pack(tT, eye):
    n = tT.shape[1]
    grid = pl.cdiv(n, PACK_LANES)
    return pl.pallas_call(
        _pack_body,
        grid=(grid,),
        in_specs=[pl.BlockSpec((EMBED_DIM, PACK_LANES), lambda g: (0, g)),
                  pl.BlockSpec((EMBED_DIM, EMBED_DIM), lambda g: (0, 0))],
        out_specs=pl.BlockSpec((PACK_ROWS, 128), lambda g: (g, 0)),
        out_shape=jax.ShapeDtypeStruct((grid * PACK_ROWS, 128), jnp.float32),
    )(tT, eye)


def _sc_gather1(packed, pidx):
    mesh = plsc.VectorSubcoreMesh(core_axis_name="core", subcore_axis_name="subcore")

    @functools.partial(
        pl.kernel,
        out_type=jax.ShapeDtypeStruct((BATCH, 128), jnp.float32),
        mesh=mesh,
        scratch_types=[pltpu.VMEM((BPW,), jnp.int32),
                       pltpu.VMEM((BPW, 128), jnp.float32),
                       pltpu.SemaphoreType.DMA])
    def gather_kernel(tbl, idx, out, idx_v, rows, sem):
        cid = lax.axis_index("core")
        sid = lax.axis_index("subcore")
        base = (sid * 2 + cid) * BPW
        pltpu.sync_copy(idx.at[pl.ds(base, BPW)], idx_v)
        copies = []
        for j in range(BPW // GW):
            copies.append(pltpu.async_copy(
                tbl.at[idx_v.at[pl.ds(j * GW, GW)]],
                rows.at[pl.ds(j * GW, GW), :], sem))
        for c in copies:
            c.wait()
        pltpu.sync_copy(rows, out.at[pl.ds(base, BPW), :])

    return gather_kernel(packed, pidx)


def _mlp_body(u_ref, r_ref, i_ref, n_ref, qu_ref, qr_ref, qi_ref, qn_ref,
              w1_ref, b1_ref, w2_ref, b2_ref, o_ref):
    embs = []
    for x_ref, q_ref in ((u_ref, qu_ref), (r_ref, qr_ref),
                         (i_ref, qi_ref), (n_ref, qn_ref)):
        v = x_ref[...]
        odd = q_ref[...] == 1
        embs.append(jnp.where(odd, v[:, EMBED_DIM:], v[:, :EMBED_DIM]))
    x = jnp.concatenate(embs, axis=1).astype(jnp.bfloat16)
    w1 = w1_ref[...].astype(jnp.bfloat16)
    h = jnp.dot(x, w1, preferred_element_type=jnp.float32) + b1_ref[...]
    h = jnp.maximum(h, 0.0).astype(jnp.bfloat16)
    w2 = w2_ref[...].astype(jnp.bfloat16)
    o_ref[...] = jnp.dot(h, w2, preferred_element_type=jnp.float32) + b2_ref[...]


def _tc_mlp(pairs, quarters, W1, b1, W2, b2):
    d4, d2, d1 = 4 * EMBED_DIM, 2 * EMBED_DIM, EMBED_DIM
    row_spec = pl.BlockSpec((MLP_BLOCK, 128), lambda g: (g, 0))
    q_spec = pl.BlockSpec((MLP_BLOCK, 1), lambda g: (g, 0))
    return pl.pallas_call(
        _mlp_body,
        grid=(BATCH // MLP_BLOCK,),
        in_specs=[
            row_spec, row_spec, row_spec, row_spec,
            q_spec, q_spec, q_spec, q_spec,
            pl.BlockSpec((d4, d2), lambda g: (0, 0)),
            pl.BlockSpec((1, d2), lambda g: (0, 0)),
            pl.BlockSpec((d2, d1), lambda g: (0, 0)),
            pl.BlockSpec((1, d1), lambda g: (0, 0)),
        ],
        out_specs=pl.BlockSpec((MLP_BLOCK, d1), lambda g: (g, 0)),
        out_shape=jax.ShapeDtypeStruct((BATCH, d1), jnp.float32),
    )(*pairs, *quarters, W1, b1.reshape(1, d2), W2, b2.reshape(1, d1))


@jax.jit
def kernel(user_idx, recipe_idx, ingredient_idx, nutrition_idx,
           user_table, recipe_table, ingredient_table, nutrition_table,
           W1, b1, W2, b2):
    eye = jnp.eye(EMBED_DIM, dtype=jnp.bfloat16)
    quarters, pairs = [], []
    for idx, tbl in ((user_idx, user_table), (recipe_idx, recipe_table),
                     (ingredient_idx, ingredient_table),
                     (nutrition_idx, nutrition_table)):
        r = idx.astype(jnp.int32)
        blk = r // PACK_LANES
        off = r % PACK_LANES
        quarters.append((off // PACK_ROWS).reshape(BATCH, 1))
        packed = _tc_pack(tbl.T, eye)
        pairs.append(_sc_gather1(packed, blk * PACK_ROWS + off % PACK_ROWS))
    return _tc_mlp(pairs, quarters, W1, b1, W2, b2)


# speedup vs baseline: 2.8426x; 1.1711x over previous
import functools

import jax
import jax.numpy as jnp
from jax import lax
from jax.experimental import pallas as pl
from jax.experimental.pallas import tpu as pltpu
from jax.experimental.pallas import tpu_sc as plsc

BATCH = 16384
EMBED_DIM = 64
HALF = EMBED_DIM // 2
PACK_LANES = 16384
PACK_ROWS = PACK_LANES // 4
TOPMASK = -65536
NUM_WORKERS = 32
BPW = BATCH // NUM_WORKERS
GW = 128
MLP_BLOCK = 2048
def _pack_body(x_ref, eye_ref, o_ref):
    x = x_ref[...].astype(jnp.bfloat16)
    eye = eye_ref[...]
    dn = (((0,), (0,)), ((), ()))
    xt = lax.dot_general(x, eye, dn, preferred_element_type=jnp.float32)
    b = lax.bitcast_convert_type(xt, jnp.int32)
    half = PACK_LANES // 2
    v = lax.shift_right_logical(b[:half], 16) | b[half:]
    o_ref[...] = jnp.concatenate([v[:PACK_ROWS], v[PACK_ROWS:]], axis=1)


def _tc_pack(tT, eye):
    n = tT.shape[1]
    grid = pl.cdiv(n, PACK_LANES)
    return pl.pallas_call(
        _pack_body,
        grid=(grid,),
        in_specs=[pl.BlockSpec((EMBED_DIM, PACK_LANES), lambda g: (0, g)),
                  pl.BlockSpec((EMBED_DIM, EMBED_DIM), lambda g: (0, 0))],
        out_specs=pl.BlockSpec((PACK_ROWS, 128), lambda g: (g, 0)),
        out_shape=jax.ShapeDtypeStruct((grid * PACK_ROWS, 128), jnp.int32),
    )(tT, eye)


def _sc_gather1(packed, pidx):
    mesh = plsc.VectorSubcoreMesh(core_axis_name="core", subcore_axis_name="subcore")

    @functools.partial(
        pl.kernel,
        out_type=jax.ShapeDtypeStruct((BATCH, 128), jnp.int32),
        mesh=mesh,
        scratch_types=[pltpu.VMEM((BPW,), jnp.int32),
                       pltpu.VMEM((BPW, 128), jnp.int32),
                       pltpu.SemaphoreType.DMA])
    def gather_kernel(tbl, idx, out, idx_v, rows, sem):
        cid = lax.axis_index("core")
        sid = lax.axis_index("subcore")
        base = (sid * 2 + cid) * BPW
        pltpu.sync_copy(idx.at[pl.ds(base, BPW)], idx_v)
        copies = []
        for j in range(BPW // GW):
            copies.append(pltpu.async_copy(
                tbl.at[idx_v.at[pl.ds(j * GW, GW)]],
                rows.at[pl.ds(j * GW, GW), :], sem))
        for c in copies:
            c.wait()
        pltpu.sync_copy(rows, out.at[pl.ds(base, BPW), :])

    return gather_kernel(packed, pidx)


def _mlp_body(u_ref, r_ref, i_ref, n_ref, qu_ref, qr_ref, qi_ref, qn_ref,
              w1_ref, b1_ref, w2_ref, b2_ref, o_ref):
    embs = []
    for x_ref, q_ref in ((u_ref, qu_ref), (r_ref, qr_ref),
                         (i_ref, qi_ref), (n_ref, qn_ref)):
        v = x_ref[...]
        q = q_ref[...]
        vsel = jnp.where((q & 1) == 1, v[:, EMBED_DIM:], v[:, :EMBED_DIM])
        bits = jnp.where(q >= 2, vsel & TOPMASK, lax.shift_left(vsel, 16))
        embs.append(lax.bitcast_convert_type(bits, jnp.float32))
    x = jnp.concatenate(embs, axis=1).astype(jnp.bfloat16)
    w1 = w1_ref[...].astype(jnp.bfloat16)
    h = jnp.dot(x, w1, preferred_element_type=jnp.float32) + b1_ref[...]
    h = jnp.maximum(h, 0.0).astype(jnp.bfloat16)
    w2 = w2_ref[...].astype(jnp.bfloat16)
    o_ref[...] = jnp.dot(h, w2, preferred_element_type=jnp.float32) + b2_ref[...]


def _tc_mlp(pairs, quarters, W1, b1, W2, b2):
    d4, d2, d1 = 4 * EMBED_DIM, 2 * EMBED_DIM, EMBED_DIM
    row_spec = pl.BlockSpec((MLP_BLOCK, 128), lambda g: (g, 0))
    q_spec = pl.BlockSpec((MLP_BLOCK, 1), lambda g: (g, 0))
    return pl.pallas_call(
        _mlp_body,
        grid=(BATCH // MLP_BLOCK,),
        in_specs=[
            row_spec, row_spec, row_spec, row_spec,
            q_spec, q_spec, q_spec, q_spec,
            pl.BlockSpec((d4, d2), lambda g: (0, 0)),
            pl.BlockSpec((1, d2), lambda g: (0, 0)),
            pl.BlockSpec((d2, d1), lambda g: (0, 0)),
            pl.BlockSpec((1, d1), lambda g: (0, 0)),
        ],
        out_specs=pl.BlockSpec((MLP_BLOCK, d1), lambda g: (g, 0)),
        out_shape=jax.ShapeDtypeStruct((BATCH, d1), jnp.float32),
    )(*pairs, *quarters, W1, b1.reshape(1, d2), W2, b2.reshape(1, d1))


@jax.jit
def kernel(user_idx, recipe_idx, ingredient_idx, nutrition_idx,
           user_table, recipe_table, ingredient_table, nutrition_table,
           W1, b1, W2, b2):
    eye = jnp.eye(EMBED_DIM, dtype=jnp.bfloat16)
    quarters, pairs = [], []
    for idx, tbl in ((user_idx, user_table), (recipe_idx, recipe_table),
                     (ingredient_idx, ingredient_table),
                     (nutrition_idx, nutrition_table)):
        r = idx.astype(jnp.int32)
        blk = r // PACK_LANES
        off = r % PACK_LANES
        quarters.append((off // PACK_ROWS).reshape(BATCH, 1))
        packed = _tc_pack(tbl.T, eye)
        pairs.append(_sc_gather1(packed, blk * PACK_ROWS + off % PACK_ROWS))
    return _tc_mlp(pairs, quarters, W1, b1, W2, b2)


# speedup vs baseline: 3.1415x; 1.1052x over previous
import functools

import jax
import jax.numpy as jnp
from jax import lax
from jax.experimental import pallas as pl
from jax.experimental.pallas import tpu as pltpu
from jax.experimental.pallas import tpu_sc as plsc

BATCH = 16384
EMBED_DIM = 64
HALF = EMBED_DIM // 2
PACK_LANES = 32768
PACK_ROWS = PACK_LANES // 4
TOPMASK = -65536
NUM_WORKERS = 32
BPW = BATCH // NUM_WORKERS
GW = 128
MLP_BLOCK = 2048
def _pack_body(x_ref, eye_ref, o_ref):
    x = x_ref[...].astype(jnp.bfloat16)
    eye = eye_ref[...]
    dn = (((0,), (0,)), ((), ()))
    xt = lax.dot_general(x, eye, dn, preferred_element_type=jnp.float32)
    b = lax.bitcast_convert_type(xt, jnp.int32)
    half = PACK_LANES // 2
    v = lax.shift_right_logical(b[:half], 16) | b[half:]
    o_ref[...] = jnp.concatenate([v[:PACK_ROWS], v[PACK_ROWS:]], axis=1)


def _tc_pack(tT, eye):
    n = tT.shape[1]
    grid = pl.cdiv(n, PACK_LANES)
    return pl.pallas_call(
        _pack_body,
        grid=(grid,),
        in_specs=[pl.BlockSpec((EMBED_DIM, PACK_LANES), lambda g: (0, g)),
                  pl.BlockSpec((EMBED_DIM, EMBED_DIM), lambda g: (0, 0))],
        out_specs=pl.BlockSpec((PACK_ROWS, 128), lambda g: (g, 0)),
        out_shape=jax.ShapeDtypeStruct((grid * PACK_ROWS, 128), jnp.int32),
    )(tT, eye)


def _sc_gather1(packed, pidx):
    mesh = plsc.VectorSubcoreMesh(core_axis_name="core", subcore_axis_name="subcore")

    @functools.partial(
        pl.kernel,
        out_type=jax.ShapeDtypeStruct((BATCH, 128), jnp.int32),
        mesh=mesh,
        scratch_types=[pltpu.VMEM((BPW,), jnp.int32),
                       pltpu.VMEM((BPW, 128), jnp.int32),
                       pltpu.SemaphoreType.DMA])
    def gather_kernel(tbl, idx, out, idx_v, rows, sem):
        cid = lax.axis_index("core")
        sid = lax.axis_index("subcore")
        base = (sid * 2 + cid) * BPW
        pltpu.sync_copy(idx.at[pl.ds(base, BPW)], idx_v)
        copies = []
        for j in range(BPW // GW):
            copies.append(pltpu.async_copy(
                tbl.at[idx_v.at[pl.ds(j * GW, GW)]],
                rows.at[pl.ds(j * GW, GW), :], sem))
        for c in copies:
            c.wait()
        pltpu.sync_copy(rows, out.at[pl.ds(base, BPW), :])

    return gather_kernel(packed, pidx)


def _mlp_body(u_ref, r_ref, i_ref, n_ref, qu_ref, qr_ref, qi_ref, qn_ref,
              w1_ref, b1_ref, w2_ref, b2_ref, o_ref):
    embs = []
    for x_ref, q_ref in ((u_ref, qu_ref), (r_ref, qr_ref),
                         (i_ref, qi_ref), (n_ref, qn_ref)):
        v = x_ref[...]
        q = q_ref[...]
        vsel = jnp.where((q & 1) == 1, v[:, EMBED_DIM:], v[:, :EMBED_DIM])
        bits = jnp.where(q >= 2, vsel & TOPMASK, lax.shift_left(vsel, 16))
        embs.append(lax.bitcast_convert_type(bits, jnp.float32))
    x = jnp.concatenate(embs, axis=1).astype(jnp.bfloat16)
    w1 = w1_ref[...].astype(jnp.bfloat16)
    h = jnp.dot(x, w1, preferred_element_type=jnp.float32) + b1_ref[...]
    h = jnp.maximum(h, 0.0).astype(jnp.bfloat16)
    w2 = w2_ref[...].astype(jnp.bfloat16)
    o_ref[...] = lax.dot_general(w2, h, (((0,), (1,)), ((), ())),
                                 preferred_element_type=jnp.float32) + b2_ref[...]


def _tc_mlp(pairs, quarters, W1, b1, W2, b2):
    d4, d2, d1 = 4 * EMBED_DIM, 2 * EMBED_DIM, EMBED_DIM
    row_spec = pl.BlockSpec((MLP_BLOCK, 128), lambda g: (g, 0))
    q_spec = pl.BlockSpec((MLP_BLOCK, 1), lambda g: (g, 0))
    return pl.pallas_call(
        _mlp_body,
        grid=(BATCH // MLP_BLOCK,),
        in_specs=[
            row_spec, row_spec, row_spec, row_spec,
            q_spec, q_spec, q_spec, q_spec,
            pl.BlockSpec((d4, d2), lambda g: (0, 0)),
            pl.BlockSpec((1, d2), lambda g: (0, 0)),
            pl.BlockSpec((d2, d1), lambda g: (0, 0)),
            pl.BlockSpec((d1, 1), lambda g: (0, 0)),
        ],
        out_specs=pl.BlockSpec((d1, MLP_BLOCK), lambda g: (0, g)),
        out_shape=jax.ShapeDtypeStruct((d1, BATCH), jnp.float32),
    )(*pairs, *quarters, W1, b1.reshape(1, d2), W2, b2.reshape(d1, 1))


@jax.jit
def kernel(user_idx, recipe_idx, ingredient_idx, nutrition_idx,
           user_table, recipe_table, ingredient_table, nutrition_table,
           W1, b1, W2, b2):
    eye = jnp.eye(EMBED_DIM, dtype=jnp.bfloat16)
    quarters, pairs = [], []
    for idx, tbl in ((user_idx, user_table), (recipe_idx, recipe_table),
                     (ingredient_idx, ingredient_table),
                     (nutrition_idx, nutrition_table)):
        r = idx.astype(jnp.int32)
        blk = r // PACK_LANES
        off = r % PACK_LANES
        quarters.append((off // PACK_ROWS).reshape(BATCH, 1))
        packed = _tc_pack(tbl.T, eye)
        pairs.append(_sc_gather1(packed, blk * PACK_ROWS + off % PACK_ROWS))
    return _tc_mlp(pairs, quarters, W1, b1, W2, b2).T
